# Initial kernel scaffold; baseline (speedup 1.0000x reference)
import jax
import jax.numpy as jnp
from jax.experimental import pallas as pl


def kernel(x):
    raise NotImplementedError("write your pallas kernel here")


# speedup vs baseline: 9.8125x; 9.8125x over previous
import jax
import jax.numpy as jnp
from jax.experimental import pallas as pl

_N = 1024
_K = 512
_ROWS = 256
_ITERS = 16


def _body(x_ref, o_ref):
    xb = x_ref[...]
    lo = jnp.min(xb, axis=-1, keepdims=True) - 1.0
    hi = jnp.max(xb, axis=-1, keepdims=True)
    for _ in range(_ITERS):
        mid = 0.5 * (lo + hi)
        cnt = jnp.sum(jnp.where(xb > mid, 1.0, 0.0), axis=-1, keepdims=True)
        ge = cnt >= _K
        lo = jnp.where(ge, mid, lo)
        hi = jnp.where(ge, hi, mid)
    m = 0.5 * (lo + hi)
    s = jnp.sum(jnp.maximum(xb - m, 0.0), axis=-1) + _K * m[:, 0]
    o_ref[...] = s * (1.0 / _K)


def kernel(x):
    b, c, h, w = x.shape
    rows = b * c
    x2 = x.reshape(rows, h * w)
    out = pl.pallas_call(
        _body,
        grid=(rows // _ROWS,),
        in_specs=[pl.BlockSpec((_ROWS, _N), lambda i: (i, 0))],
        out_specs=pl.BlockSpec((_ROWS,), lambda i: (i,)),
        out_shape=jax.ShapeDtypeStruct((rows,), jnp.float32),
    )(x2)
    return out.reshape(b, c)
